# baseline (speedup 1.0000x reference)
import functools

import jax
import jax.numpy as jnp
from jax import lax
from jax.experimental import pallas as pl
from jax.experimental.pallas import tpu as pltpu
from jax.experimental.pallas import tpu_sc as plsc

_N = 10000
_E = 160000
_IN = 256
_H = 8
_D = 64
_HD = _H * _D
_ALPHA = 0.2
_ROWW = _HD + 16

_NC = 2
_NS = 16
_NPAD = 10240
_NCH = 8
_CPS = _NCH // _NC
_CHUNK = _NPAD // _NCH
_RPT = _CHUNK // _NS
_EPT = _E // _NS
_B = 48
_CBIG = _EPT + 48


def _tc_body(x_ref, wt_ref, al_ref, ar_ref, fts_ref, a2_ref):
    ft = jnp.dot(x_ref[...], wt_ref[...], preferred_element_type=jnp.float32)
    a1 = jnp.dot(ft, al_ref[...], preferred_element_type=jnp.float32)
    a2 = jnp.dot(ft, ar_ref[...], preferred_element_type=jnp.float32)
    fts_ref[...] = jnp.concatenate([ft, a1], axis=1)
    a2_ref[...] = a2


def _tc_project(x, wt, al16, ar16):
    rows = 400
    return pl.pallas_call(
        _tc_body,
        grid=(_N // rows,),
        in_specs=[
            pl.BlockSpec((rows, _IN), lambda i: (i, 0)),
            pl.BlockSpec((_IN, _HD), lambda i: (0, 0)),
            pl.BlockSpec((_HD, 16), lambda i: (0, 0)),
            pl.BlockSpec((_HD, 16), lambda i: (0, 0)),
        ],
        out_specs=[
            pl.BlockSpec((rows, _ROWW), lambda i: (i, 0)),
            pl.BlockSpec((rows, 16), lambda i: (i, 0)),
        ],
        out_shape=[
            jax.ShapeDtypeStruct((_N, _ROWW), jnp.float32),
            jax.ShapeDtypeStruct((_N, 16), jnp.float32),
        ],
    )(x, wt, al16, ar16)


def _pk_body(ei_ref, out_ref):
    out_ref[...] = lax.shift_left(ei_ref[0:1, :], 14) | ei_ref[1:2, :]


def _pack_edges(ei):
    cols = _E // 10
    return pl.pallas_call(
        _pk_body,
        grid=(10,),
        in_specs=[pl.BlockSpec((2, cols), lambda i: (0, i))],
        out_specs=pl.BlockSpec((1, cols), lambda i: (0, i)),
        out_shape=jax.ShapeDtypeStruct((1, _E), jnp.int32),
    )(ei)


_sc_mesh = plsc.VectorSubcoreMesh(
    core_axis_name="c", subcore_axis_name="s", num_cores=_NC, num_subcores=_NS
)


@functools.partial(
    pl.kernel,
    out_type=jax.ShapeDtypeStruct((_N, _HD), jnp.float32),
    mesh=_sc_mesh,
    compiler_params=pltpu.CompilerParams(
        needs_layout_passes=False, use_tc_tiling_on_sc=False
    ),
    scratch_types=[
        pltpu.VMEM((_EPT,), jnp.int32),
        pltpu.VMEM((_CBIG,), jnp.int32),
        pltpu.VMEM((2, _B), jnp.int32),
        pltpu.VMEM((2, _B), jnp.int32),
        pltpu.VMEM((2, _B), jnp.int32),
        pltpu.VMEM((_B, _ROWW), jnp.float32),
        pltpu.VMEM((_B, _ROWW), jnp.float32),
        pltpu.VMEM((_B, 16), jnp.float32),
        pltpu.VMEM((_B, 16), jnp.float32),
        pltpu.VMEM((4, _ROWW), jnp.float32),
        pltpu.VMEM((8, _ROWW), jnp.float32),
        pltpu.VMEM((8, _HD), jnp.float32),
        pltpu.VMEM_SHARED((_CHUNK, _ROWW), jnp.float32),
        pltpu.SemaphoreType.DMA,
        pltpu.SemaphoreType.DMA,
        pltpu.SemaphoreType.DMA,
        pltpu.SemaphoreType.DMA,
    ],
)
def _sc_edge(fts, a2d, epk, out, epkv, cbig, gsrc, gdst, sidx, rows0, rows1,
             a2b0, a2b1, zbuf, orow, outw, acc, semg0, semg1, sems0, sems1):
    cid = lax.axis_index("c")
    sid = lax.axis_index("s")
    rows_s = (rows0, rows1)
    a2b_s = (a2b0, a2b1)
    semg_s = (semg0, semg1)
    sems_s = (sems0, sems1)

    pltpu.sync_copy(epk.at[pl.ds(sid * _EPT, _EPT)], epkv)

    zv = jnp.zeros((16,), jnp.float32)

    def zrow(i, _):
        for j in range(_ROWW // 16):
            zbuf[i, pl.ds(j * 16, 16)] = zv
        return 0

    lax.fori_loop(0, 4, zrow, 0)

    rbase = sid * _RPT

    for cc in range(_CPS):
        base = (cid * _CPS + cc) * _CHUNK

        def zacc(r, _):
            pltpu.sync_copy(zbuf, acc.at[pl.ds(rbase + r * 4, 4)])
            return 0

        lax.fori_loop(0, _RPT // 4, zacc, 0)
        plsc.subcore_barrier()

        def scan_step(g, cnt):
            ep = epkv[pl.ds(g * 16, 16)]
            dv = ep & 16383
            m = (dv >= base) & (dv < base + _CHUNK)
            lanes = lax.iota(jnp.int32, 16)
            key = jnp.where(m, lanes, lanes + 16)
            _, eps = lax.sort((key, ep), num_keys=1)
            cbig[pl.ds(cnt, 16)] = eps
            return cnt + plsc.all_reduce_population_count(m)[0]

        count = lax.fori_loop(0, _EPT // 16, scan_step, jnp.int32(0))
        nb = (count + _B - 1) // _B

        def prep(g, p):
            cb = jnp.minimum(count - g * _B, _B)
            for k in range(_B // 16):
                gi = lax.iota(jnp.int32, 16) + (k * 16)
                pv = cbig[pl.ds(g * _B + k * 16, 16)]
                valid = gi < cb
                pv = jnp.where(valid, pv, 0)
                dvg = pv & 16383
                gsrc[p, pl.ds(k * 16, 16)] = lax.shift_right_logical(pv, 14)
                gdst[p, pl.ds(k * 16, 16)] = dvg
                sidx[p, pl.ds(k * 16, 16)] = jnp.where(valid, dvg - base, 0)

        def gather_start(p):
            pltpu.async_copy(fts.at[gsrc.at[p]], rows_s[p], semg_s[p])
            pltpu.async_copy(a2d.at[gdst.at[p]], a2b_s[p], semg_s[p])

        def gather_wait(p):
            pltpu.make_async_copy(fts.at[gsrc.at[p]], rows_s[p],
                                  semg_s[p]).wait()
            pltpu.make_async_copy(a2d.at[gdst.at[p]], a2b_s[p],
                                  semg_s[p]).wait()

        def scatter_start(p):
            pltpu.async_copy(rows_s[p], acc.at[sidx.at[p]], sems_s[p],
                             add=True)

        def scatter_wait(p):
            pltpu.make_async_copy(rows_s[p], acc.at[sidx.at[p]],
                                  sems_s[p]).wait()

        def compute(g, p):
            rws = rows_s[p]
            a2w = a2b_s[p]
            cb = jnp.minimum(count - g * _B, _B)

            @plsc.parallel_loop(0, _B, unroll=4)
            def edge(b):
                a1v = rws[b, pl.ds(_HD, 16)]
                a2v = a2w[b, pl.ds(0, 16)]
                ev = a1v + a2v
                e = jnp.where(ev >= 0.0, ev, _ALPHA * ev)
                s = jnp.exp(e) * jnp.where(b < cb, 1.0, 0.0)
                for h in range(_H):
                    sh = s[h]
                    for q in range(4):
                        j = h * 4 + q
                        rws[b, pl.ds(j * 16, 16)] = rws[b, pl.ds(j * 16, 16)] * sh
                rws[b, pl.ds(_HD, 16)] = s

        @pl.when(nb >= 1)
        def _():
            prep(jnp.int32(0), 0)
            gather_start(0)

        def outer(i, _):
            for p in range(2):
                g = 2 * i + p

                @pl.when(g < nb)
                def _():
                    @pl.when(g + 1 < nb)
                    def _():
                        @pl.when(g >= 1)
                        def _():
                            scatter_wait(1 - p)

                        prep(g + 1, 1 - p)
                        gather_start(1 - p)

                    gather_wait(p)
                    compute(g, p)
                    scatter_start(p)
            return 0

        lax.fori_loop(0, (nb + 1) // 2, outer, 0)

        @pl.when(nb >= 1)
        def _():
            scatter_wait(0)

        @pl.when(nb >= 2)
        def _():
            scatter_wait(1)

        plsc.subcore_barrier()

        def egroup(g, _):
            r0 = rbase + g * 8

            @pl.when(base + r0 < _N)
            def _():
                pltpu.sync_copy(acc.at[pl.ds(r0, 8)], orow)

                @plsc.parallel_loop(0, 8, unroll=2)
                def erow(r):
                    for h in range(_H):
                        zsp = plsc.load_gather(
                            orow,
                            [jnp.full((16,), r, jnp.int32),
                             jnp.full((16,), _HD + h, jnp.int32)],
                        )
                        den = jnp.where(zsp == 0.0, 1.0, zsp)
                        for q in range(4):
                            col = h * 64 + q * 16
                            outw[r, pl.ds(col, 16)] = orow[r, pl.ds(col, 16)] / den

                pltpu.sync_copy(outw, out.at[pl.ds(base + r0, 8)])
            return 0

        lax.fori_loop(0, _RPT // 8, egroup, 0)
        plsc.subcore_barrier()


def kernel(x, edge_index, W, attn_l, attn_r):
    wt = W.T
    al = attn_l[:, :, 0]
    ar = attn_r[:, :, 0]
    eye = jnp.eye(_H, dtype=jnp.float32)
    al8 = (eye[:, None, :] * al[:, :, None]).reshape(_HD, _H)
    ar8 = (eye[:, None, :] * ar[:, :, None]).reshape(_HD, _H)
    al16 = jnp.concatenate([al8, al8], axis=1)
    ar16 = jnp.concatenate([ar8, ar8], axis=1)
    fts, a2d = _tc_project(x, wt, al16, ar16)
    epk = _pack_edges(edge_index).reshape(_E)
    out = _sc_edge(fts, a2d, epk)
    return out.reshape(_N, _H, _D)


# speedup vs baseline: 1.0798x; 1.0798x over previous
import functools

import jax
import jax.numpy as jnp
from jax import lax
from jax.experimental import pallas as pl
from jax.experimental.pallas import tpu as pltpu
from jax.experimental.pallas import tpu_sc as plsc

_N = 10000
_E = 160000
_IN = 256
_H = 8
_D = 64
_HD = _H * _D
_ALPHA = 0.2
_ROWW = _HD + 16

_NC = 2
_NS = 16
_NPAD = 10240
_NCH = 8
_CPS = _NCH // _NC
_CHUNK = _NPAD // _NCH
_RPT = _CHUNK // _NS
_EPT = _E // _NS
_B = 32
_CBIG = _EPT + 48


def _tc_body(x_ref, wt_ref, al_ref, ar_ref, fts_ref, a2_ref):
    ft = jnp.dot(x_ref[...], wt_ref[...], preferred_element_type=jnp.float32)
    a1 = jnp.dot(ft, al_ref[...], preferred_element_type=jnp.float32)
    a2 = jnp.dot(ft, ar_ref[...], preferred_element_type=jnp.float32)
    fts_ref[...] = jnp.concatenate([ft, a1], axis=1)
    a2_ref[...] = a2


def _tc_project(x, wt, al16, ar16):
    rows = 400
    return pl.pallas_call(
        _tc_body,
        grid=(_N // rows,),
        in_specs=[
            pl.BlockSpec((rows, _IN), lambda i: (i, 0)),
            pl.BlockSpec((_IN, _HD), lambda i: (0, 0)),
            pl.BlockSpec((_HD, 16), lambda i: (0, 0)),
            pl.BlockSpec((_HD, 16), lambda i: (0, 0)),
        ],
        out_specs=[
            pl.BlockSpec((rows, _ROWW), lambda i: (i, 0)),
            pl.BlockSpec((rows, 16), lambda i: (i, 0)),
        ],
        out_shape=[
            jax.ShapeDtypeStruct((_N, _ROWW), jnp.float32),
            jax.ShapeDtypeStruct((_N, 16), jnp.float32),
        ],
    )(x, wt, al16, ar16)


def _pk_body(ei_ref, out_ref):
    out_ref[...] = lax.shift_left(ei_ref[0:1, :], 14) | ei_ref[1:2, :]


def _pack_edges(ei):
    cols = _E // 10
    return pl.pallas_call(
        _pk_body,
        grid=(10,),
        in_specs=[pl.BlockSpec((2, cols), lambda i: (0, i))],
        out_specs=pl.BlockSpec((1, cols), lambda i: (0, i)),
        out_shape=jax.ShapeDtypeStruct((1, _E), jnp.int32),
    )(ei)


_sc_mesh = plsc.VectorSubcoreMesh(
    core_axis_name="c", subcore_axis_name="s", num_cores=_NC, num_subcores=_NS
)


@functools.partial(
    pl.kernel,
    out_type=jax.ShapeDtypeStruct((_N, _HD), jnp.float32),
    mesh=_sc_mesh,
    compiler_params=pltpu.CompilerParams(
        needs_layout_passes=False, use_tc_tiling_on_sc=False
    ),
    scratch_types=[
        pltpu.VMEM((_EPT,), jnp.int32),
        pltpu.VMEM((_CBIG,), jnp.int32),
        pltpu.VMEM((2, _B), jnp.int32),
        pltpu.VMEM((2, _B), jnp.int32),
        pltpu.VMEM((2, _B), jnp.int32),
        pltpu.VMEM((_B, _ROWW), jnp.float32),
        pltpu.VMEM((_B, _ROWW), jnp.float32),
        pltpu.VMEM((_B, 16), jnp.float32),
        pltpu.VMEM((_B, 16), jnp.float32),
        pltpu.VMEM((4, _ROWW), jnp.float32),
        pltpu.VMEM((8, _ROWW), jnp.float32),
        pltpu.VMEM((8, _HD), jnp.float32),
        pltpu.VMEM_SHARED((_CHUNK, _ROWW), jnp.float32),
        pltpu.SemaphoreType.DMA,
        pltpu.SemaphoreType.DMA,
        pltpu.SemaphoreType.DMA,
        pltpu.SemaphoreType.DMA,
    ],
)
def _sc_edge(fts, a2d, epk, out, epkv, cbig, gsrc, gdst, sidx, rows0, rows1,
             a2b0, a2b1, zbuf, orow, outw, acc, semg0, semg1, sems0, sems1):
    cid = lax.axis_index("c")
    sid = lax.axis_index("s")
    rows_s = (rows0, rows1)
    a2b_s = (a2b0, a2b1)
    semg_s = (semg0, semg1)
    sems_s = (sems0, sems1)

    pltpu.sync_copy(epk.at[pl.ds(sid * _EPT, _EPT)], epkv)

    zv = jnp.zeros((16,), jnp.float32)

    def zrow(i, _):
        for j in range(_ROWW // 16):
            zbuf[i, pl.ds(j * 16, 16)] = zv
        return 0

    lax.fori_loop(0, 4, zrow, 0)

    rbase = sid * _RPT

    for cc in range(_CPS):
        base = (cid * _CPS + cc) * _CHUNK

        def zacc(r, _):
            pltpu.sync_copy(zbuf, acc.at[pl.ds(rbase + r * 4, 4)])
            return 0

        lax.fori_loop(0, _RPT // 4, zacc, 0)
        plsc.subcore_barrier()

        def scan_step(g, cnt):
            ep = epkv[pl.ds(g * 16, 16)]
            dv = ep & 16383
            m = (dv >= base) & (dv < base + _CHUNK)
            lanes = lax.iota(jnp.int32, 16)
            key = jnp.where(m, lanes, lanes + 16)
            _, eps = lax.sort((key, ep), num_keys=1)
            cbig[pl.ds(cnt, 16)] = eps
            return cnt + plsc.all_reduce_population_count(m)[0]

        count = lax.fori_loop(0, _EPT // 16, scan_step, jnp.int32(0))
        nb = (count + _B - 1) // _B

        def prep(g, p):
            cb = jnp.minimum(count - g * _B, _B)
            for k in range(_B // 16):
                gi = lax.iota(jnp.int32, 16) + (k * 16)
                pv = cbig[pl.ds(g * _B + k * 16, 16)]
                valid = gi < cb
                pv = jnp.where(valid, pv, 0)
                dvg = pv & 16383
                gsrc[p, pl.ds(k * 16, 16)] = lax.shift_right_logical(pv, 14)
                gdst[p, pl.ds(k * 16, 16)] = dvg
                sidx[p, pl.ds(k * 16, 16)] = jnp.where(valid, dvg - base, 0)

        def gather_start(p):
            pltpu.async_copy(fts.at[gsrc.at[p]], rows_s[p], semg_s[p])
            pltpu.async_copy(a2d.at[gdst.at[p]], a2b_s[p], semg_s[p])

        def gather_wait(p):
            pltpu.make_async_copy(fts.at[gsrc.at[p]], rows_s[p],
                                  semg_s[p]).wait()
            pltpu.make_async_copy(a2d.at[gdst.at[p]], a2b_s[p],
                                  semg_s[p]).wait()

        def scatter_start(p):
            pltpu.async_copy(rows_s[p], acc.at[sidx.at[p]], sems_s[p],
                             add=True)

        def scatter_wait(p):
            pltpu.make_async_copy(rows_s[p], acc.at[sidx.at[p]],
                                  sems_s[p]).wait()

        def compute(g, p):
            rws = rows_s[p]
            a2w = a2b_s[p]
            cb = jnp.minimum(count - g * _B, _B)

            @plsc.parallel_loop(0, _B, unroll=4)
            def edge(b):
                a1v = rws[b, pl.ds(_HD, 16)]
                a2v = a2w[b, pl.ds(0, 16)]
                ev = a1v + a2v
                e = jnp.where(ev >= 0.0, ev, _ALPHA * ev)
                s = jnp.exp(e) * jnp.where(b < cb, 1.0, 0.0)
                for h in range(_H):
                    sh = s[h]
                    for q in range(4):
                        j = h * 4 + q
                        rws[b, pl.ds(j * 16, 16)] = rws[b, pl.ds(j * 16, 16)] * sh
                rws[b, pl.ds(_HD, 16)] = s

        @pl.when(nb >= 1)
        def _():
            prep(jnp.int32(0), 0)
            gather_start(0)

        def outer(i, _):
            for p in range(2):
                g = 2 * i + p

                @pl.when(g < nb)
                def _():
                    @pl.when(g + 1 < nb)
                    def _():
                        @pl.when(g >= 1)
                        def _():
                            scatter_wait(1 - p)

                        prep(g + 1, 1 - p)
                        gather_start(1 - p)

                    gather_wait(p)
                    compute(g, p)
                    scatter_start(p)
            return 0

        lax.fori_loop(0, (nb + 1) // 2, outer, 0)

        @pl.when(nb >= 1)
        def _():
            scatter_wait(0)

        @pl.when(nb >= 2)
        def _():
            scatter_wait(1)

        plsc.subcore_barrier()

        def egroup(g, _):
            r0 = rbase + g * 8

            @pl.when(base + r0 < _N)
            def _():
                pltpu.sync_copy(acc.at[pl.ds(r0, 8)], orow)

                @plsc.parallel_loop(0, 8, unroll=2)
                def erow(r):
                    for h in range(_H):
                        zsp = plsc.load_gather(
                            orow,
                            [jnp.full((16,), r, jnp.int32),
                             jnp.full((16,), _HD + h, jnp.int32)],
                        )
                        den = jnp.where(zsp == 0.0, 1.0, zsp)
                        for q in range(4):
                            col = h * 64 + q * 16
                            outw[r, pl.ds(col, 16)] = orow[r, pl.ds(col, 16)] / den

                pltpu.sync_copy(outw, out.at[pl.ds(base + r0, 8)])
            return 0

        lax.fori_loop(0, _RPT // 8, egroup, 0)
        plsc.subcore_barrier()


def kernel(x, edge_index, W, attn_l, attn_r):
    wt = W.T
    al = attn_l[:, :, 0]
    ar = attn_r[:, :, 0]
    eye = jnp.eye(_H, dtype=jnp.float32)
    al8 = (eye[:, None, :] * al[:, :, None]).reshape(_HD, _H)
    ar8 = (eye[:, None, :] * ar[:, :, None]).reshape(_HD, _H)
    al16 = jnp.concatenate([al8, al8], axis=1)
    ar16 = jnp.concatenate([ar8, ar8], axis=1)
    fts, a2d = _tc_project(x, wt, al16, ar16)
    epk = _pack_edges(edge_index).reshape(_E)
    out = _sc_edge(fts, a2d, epk)
    return out.reshape(_N, _H, _D)


# speedup vs baseline: 1.0861x; 1.0058x over previous
import functools

import jax
import jax.numpy as jnp
from jax import lax
from jax.experimental import pallas as pl
from jax.experimental.pallas import tpu as pltpu
from jax.experimental.pallas import tpu_sc as plsc

_N = 10000
_E = 160000
_IN = 256
_H = 8
_D = 64
_HD = _H * _D
_ALPHA = 0.2
_ROWW = _HD + 16
_RWI = 256 + 16

_NC = 2
_NS = 16
_NPAD = 10240
_NCH = 8
_CPS = _NCH // _NC
_CHUNK = _NPAD // _NCH
_RPT = _CHUNK // _NS
_EPT = _E // _NS
_B = 32
_CBIG = _EPT + 48


def _tc_body(x_ref, wt_ref, al_ref, ar_ref, fts_ref, a2_ref):
    ft = jnp.dot(x_ref[...], wt_ref[...], preferred_element_type=jnp.float32)
    a1 = jnp.dot(ft, al_ref[...], preferred_element_type=jnp.float32)
    a2 = jnp.dot(ft, ar_ref[...], preferred_element_type=jnp.float32)
    lob = lax.bitcast_convert_type(
        ft[:, :256].astype(jnp.bfloat16).astype(jnp.float32), jnp.int32)
    hib = lax.bitcast_convert_type(
        ft[:, 256:].astype(jnp.bfloat16).astype(jnp.float32), jnp.int32)
    packed = lax.shift_right_logical(lob, 16) | (hib & jnp.int32(-65536))
    a1i = lax.bitcast_convert_type(a1, jnp.int32)
    fts_ref[...] = jnp.concatenate([packed, a1i], axis=1)
    a2_ref[...] = a2


def _tc_project(x, wt, al16, ar16):
    rows = 400
    return pl.pallas_call(
        _tc_body,
        grid=(_N // rows,),
        in_specs=[
            pl.BlockSpec((rows, _IN), lambda i: (i, 0)),
            pl.BlockSpec((_IN, _HD), lambda i: (0, 0)),
            pl.BlockSpec((_HD, 16), lambda i: (0, 0)),
            pl.BlockSpec((_HD, 16), lambda i: (0, 0)),
        ],
        out_specs=[
            pl.BlockSpec((rows, _RWI), lambda i: (i, 0)),
            pl.BlockSpec((rows, 16), lambda i: (i, 0)),
        ],
        out_shape=[
            jax.ShapeDtypeStruct((_N, _RWI), jnp.int32),
            jax.ShapeDtypeStruct((_N, 16), jnp.float32),
        ],
    )(x, wt, al16, ar16)


def _pk_body(ei_ref, out_ref):
    out_ref[...] = lax.shift_left(ei_ref[0:1, :], 14) | ei_ref[1:2, :]


def _pack_edges(ei):
    cols = _E // 10
    return pl.pallas_call(
        _pk_body,
        grid=(10,),
        in_specs=[pl.BlockSpec((2, cols), lambda i: (0, i))],
        out_specs=pl.BlockSpec((1, cols), lambda i: (0, i)),
        out_shape=jax.ShapeDtypeStruct((1, _E), jnp.int32),
    )(ei)


_sc_mesh = plsc.VectorSubcoreMesh(
    core_axis_name="c", subcore_axis_name="s", num_cores=_NC, num_subcores=_NS
)


@functools.partial(
    pl.kernel,
    out_type=jax.ShapeDtypeStruct((_N, _HD), jnp.float32),
    mesh=_sc_mesh,
    compiler_params=pltpu.CompilerParams(
        needs_layout_passes=False, use_tc_tiling_on_sc=False
    ),
    scratch_types=[
        pltpu.VMEM((_EPT,), jnp.int32),
        pltpu.VMEM((_CBIG,), jnp.int32),
        pltpu.VMEM((2, _B), jnp.int32),
        pltpu.VMEM((2, _B), jnp.int32),
        pltpu.VMEM((2, _B), jnp.int32),
        pltpu.VMEM((_B, _RWI), jnp.int32),
        pltpu.VMEM((_B, _RWI), jnp.int32),
        pltpu.VMEM((_B, _ROWW), jnp.float32),
        pltpu.VMEM((_B, _ROWW), jnp.float32),
        pltpu.VMEM((_B, 16), jnp.float32),
        pltpu.VMEM((_B, 16), jnp.float32),
        pltpu.VMEM((4, _ROWW), jnp.float32),
        pltpu.VMEM((8, _ROWW), jnp.float32),
        pltpu.VMEM((8, _HD), jnp.float32),
        pltpu.VMEM_SHARED((_CHUNK, _ROWW), jnp.float32),
        pltpu.SemaphoreType.DMA,
        pltpu.SemaphoreType.DMA,
        pltpu.SemaphoreType.DMA,
        pltpu.SemaphoreType.DMA,
    ],
)
def _sc_edge(fts, a2d, epk, out, epkv, cbig, gsrc, gdst, sidx, rows0, rows1,
             msg0, msg1, a2b0, a2b1, zbuf, orow, outw, acc, semg0, semg1,
             sems0, sems1):
    cid = lax.axis_index("c")
    sid = lax.axis_index("s")
    rows_s = (rows0, rows1)
    msg_s = (msg0, msg1)
    a2b_s = (a2b0, a2b1)
    semg_s = (semg0, semg1)
    sems_s = (sems0, sems1)

    pltpu.sync_copy(epk.at[pl.ds(sid * _EPT, _EPT)], epkv)

    zv = jnp.zeros((16,), jnp.float32)

    def zrow(i, _):
        for j in range(_ROWW // 16):
            zbuf[i, pl.ds(j * 16, 16)] = zv
        return 0

    lax.fori_loop(0, 4, zrow, 0)

    rbase = sid * _RPT

    for cc in range(_CPS):
        base = (cid * _CPS + cc) * _CHUNK

        def zacc(r, _):
            pltpu.sync_copy(zbuf, acc.at[pl.ds(rbase + r * 4, 4)])
            return 0

        lax.fori_loop(0, _RPT // 4, zacc, 0)
        plsc.subcore_barrier()

        def scan_step(g, cnt):
            ep = epkv[pl.ds(g * 16, 16)]
            dv = ep & 16383
            m = (dv >= base) & (dv < base + _CHUNK)
            lanes = lax.iota(jnp.int32, 16)
            key = jnp.where(m, lanes, lanes + 16)
            _, eps = lax.sort((key, ep), num_keys=1)
            cbig[pl.ds(cnt, 16)] = eps
            return cnt + plsc.all_reduce_population_count(m)[0]

        count = lax.fori_loop(0, _EPT // 16, scan_step, jnp.int32(0))
        nb = (count + _B - 1) // _B

        def prep(g, p):
            cb = jnp.minimum(count - g * _B, _B)
            for k in range(_B // 16):
                gi = lax.iota(jnp.int32, 16) + (k * 16)
                pv = cbig[pl.ds(g * _B + k * 16, 16)]
                valid = gi < cb
                pv = jnp.where(valid, pv, 0)
                dvg = pv & 16383
                gsrc[p, pl.ds(k * 16, 16)] = lax.shift_right_logical(pv, 14)
                gdst[p, pl.ds(k * 16, 16)] = dvg
                sidx[p, pl.ds(k * 16, 16)] = jnp.where(valid, dvg - base, 0)

        def gather_start(p):
            pltpu.async_copy(fts.at[gsrc.at[p]], rows_s[p], semg_s[p])
            pltpu.async_copy(a2d.at[gdst.at[p]], a2b_s[p], semg_s[p])

        def gather_wait(p):
            pltpu.make_async_copy(fts.at[gsrc.at[p]], rows_s[p],
                                  semg_s[p]).wait()
            pltpu.make_async_copy(a2d.at[gdst.at[p]], a2b_s[p],
                                  semg_s[p]).wait()

        def scatter_start(p):
            pltpu.async_copy(msg_s[p], acc.at[sidx.at[p]], sems_s[p],
                             add=True)

        def scatter_wait(p):
            pltpu.make_async_copy(msg_s[p], acc.at[sidx.at[p]],
                                  sems_s[p]).wait()

        def compute(g, p):
            rws = rows_s[p]
            msw = msg_s[p]
            a2w = a2b_s[p]
            cb = jnp.minimum(count - g * _B, _B)

            @plsc.parallel_loop(0, _B, unroll=4)
            def edge(b):
                a1v = plsc.bitcast(rws[b, pl.ds(256, 16)], jnp.float32)
                a2v = a2w[b, pl.ds(0, 16)]
                ev = a1v + a2v
                e = jnp.where(ev >= 0.0, ev, _ALPHA * ev)
                s = jnp.exp(e) * jnp.where(b < cb, 1.0, 0.0)
                for i in range(16):
                    w = rws[b, pl.ds(i * 16, 16)]
                    lo = plsc.bitcast(lax.shift_left(w, 16), jnp.float32)
                    hi = plsc.bitcast(w & jnp.int32(-65536), jnp.float32)
                    msw[b, pl.ds(i * 16, 16)] = lo * s[i // 4]
                    msw[b, pl.ds(256 + i * 16, 16)] = hi * s[4 + i // 4]
                msw[b, pl.ds(_HD, 16)] = s

        @pl.when(nb >= 1)
        def _():
            prep(jnp.int32(0), 0)
            gather_start(0)

        def outer(i, _):
            for p in range(2):
                g = 2 * i + p

                @pl.when(g < nb)
                def _():
                    @pl.when(g + 1 < nb)
                    def _():
                        @pl.when(g >= 1)
                        def _():
                            scatter_wait(1 - p)

                        prep(g + 1, 1 - p)
                        gather_start(1 - p)

                    gather_wait(p)
                    compute(g, p)
                    scatter_start(p)
            return 0

        lax.fori_loop(0, (nb + 1) // 2, outer, 0)

        @pl.when(nb >= 1)
        def _():
            scatter_wait(0)

        @pl.when(nb >= 2)
        def _():
            scatter_wait(1)

        plsc.subcore_barrier()

        def egroup(g, _):
            r0 = rbase + g * 8

            @pl.when(base + r0 < _N)
            def _():
                pltpu.sync_copy(acc.at[pl.ds(r0, 8)], orow)

                @plsc.parallel_loop(0, 8, unroll=2)
                def erow(r):
                    for h in range(_H):
                        zsp = plsc.load_gather(
                            orow,
                            [jnp.full((16,), r, jnp.int32),
                             jnp.full((16,), _HD + h, jnp.int32)],
                        )
                        den = jnp.where(zsp == 0.0, 1.0, zsp)
                        for q in range(4):
                            col = h * 64 + q * 16
                            outw[r, pl.ds(col, 16)] = orow[r, pl.ds(col, 16)] / den

                pltpu.sync_copy(outw, out.at[pl.ds(base + r0, 8)])
            return 0

        lax.fori_loop(0, _RPT // 8, egroup, 0)
        plsc.subcore_barrier()


def kernel(x, edge_index, W, attn_l, attn_r):
    wt = W.T
    al = attn_l[:, :, 0]
    ar = attn_r[:, :, 0]
    eye = jnp.eye(_H, dtype=jnp.float32)
    al8 = (eye[:, None, :] * al[:, :, None]).reshape(_HD, _H)
    ar8 = (eye[:, None, :] * ar[:, :, None]).reshape(_HD, _H)
    al16 = jnp.concatenate([al8, al8], axis=1)
    ar16 = jnp.concatenate([ar8, ar8], axis=1)
    fts, a2d = _tc_project(x, wt, al16, ar16)
    epk = _pack_edges(edge_index).reshape(_E)
    out = _sc_edge(fts, a2d, epk)
    return out.reshape(_N, _H, _D)
